# Initial kernel scaffold; baseline (speedup 1.0000x reference)
import jax
import jax.numpy as jnp
from jax.experimental import pallas as pl
from jax.experimental.pallas import tpu as pltpu

_D = 64
_C = 1024
_B = 16
_T = 1024
_N = _B * _T


def _vq_body(x_ref, e_ref, q_ref, idx_ref, pplx_ref, counts_ref):
    b = pl.program_id(0)
    xb = x_ref[0]
    e = e_ref[...]
    xe = jax.lax.dot_general(xb, e, (((0,), (0,)), ((), ())),
                             preferred_element_type=jnp.float32,
                             precision=jax.lax.Precision.HIGHEST)
    x2 = jnp.sum(xb * xb, axis=0)
    e2 = jnp.sum(e * e, axis=0)
    dist = (x2[:, None] - 2.0 * xe) + e2[None, :]
    idx = jnp.argmin(dist, axis=1).astype(jnp.int32)
    idx_ref[0, 0] = idx
    oh_f = (jax.lax.broadcasted_iota(jnp.int32, (_T, _C), 1)
            == idx[:, None]).astype(jnp.float32)
    q = jax.lax.dot_general(e, oh_f, (((1,), (1,)), ((), ())),
                            preferred_element_type=jnp.float32,
                            precision=jax.lax.Precision.HIGHEST)
    q_ref[0] = xb + (q - xb)

    @pl.when(b == 0)
    def _init():
        counts_ref[...] = jnp.zeros_like(counts_ref)

    counts_ref[0, :] += jnp.sum(oh_f, axis=0)

    @pl.when(b == _B - 1)
    def _fin():
        probs = counts_ref[0, :] * (1.0 / _N)
        ent = -jnp.sum(probs * jnp.log(probs + 1e-10))
        pplx_ref[0, 0] = jnp.exp(ent)


def _make_call(interpret=False):
    return pl.pallas_call(
        _vq_body,
        grid=(_B,),
        in_specs=[
            pl.BlockSpec((1, _D, _T), lambda b: (b, 0, 0)),
            pl.BlockSpec((_D, _C), lambda b: (0, 0)),
        ],
        out_specs=[
            pl.BlockSpec((1, _D, _T), lambda b: (b, 0, 0)),
            pl.BlockSpec((1, 1, _T), lambda b: (b, 0, 0)),
            pl.BlockSpec((1, 1), lambda b: (0, 0)),
        ],
        out_shape=[
            jax.ShapeDtypeStruct((_B, _D, _T), jnp.float32),
            jax.ShapeDtypeStruct((_B, 1, _T), jnp.int32),
            jax.ShapeDtypeStruct((1, 1), jnp.float32),
        ],
        scratch_shapes=[pltpu.VMEM((1, _C), jnp.float32)],
        interpret=interpret,
    )


def kernel(x, embed):
    q, idx3, pplx = _make_call()(x, embed)
    return q, idx3.reshape(_B, _T), pplx[0, 0]


# speedup vs baseline: 1.1925x; 1.1925x over previous
import jax
import jax.numpy as jnp
from jax.experimental import pallas as pl
from jax.experimental.pallas import tpu as pltpu

_D = 64
_C = 1024
_B = 16
_T = 1024
_N = _B * _T


def _vq_body(x_ref, e_ref, q_ref, idx_ref, pplx_ref, counts_ref):
    b = pl.program_id(0)
    xb = x_ref[0]
    e = e_ref[...]
    xe = jax.lax.dot_general(xb.astype(jnp.bfloat16), e.astype(jnp.bfloat16),
                             (((0,), (0,)), ((), ())),
                             preferred_element_type=jnp.float32)
    x2 = jnp.sum(xb * xb, axis=0)
    e2 = jnp.sum(e * e, axis=0)
    dist = (x2[:, None] - 2.0 * xe) + e2[None, :]
    idx = jnp.argmin(dist, axis=1).astype(jnp.int32)
    idx_ref[0, 0] = idx
    oh_f = (jax.lax.broadcasted_iota(jnp.int32, (_T, _C), 1)
            == idx[:, None]).astype(jnp.float32)
    q = jax.lax.dot_general(e, oh_f, (((1,), (1,)), ((), ())),
                            preferred_element_type=jnp.float32,
                            precision=jax.lax.Precision.HIGHEST)
    q_ref[0] = xb + (q - xb)

    @pl.when(b == 0)
    def _init():
        counts_ref[...] = jnp.zeros_like(counts_ref)

    counts_ref[0, :] += jnp.sum(oh_f, axis=0)

    @pl.when(b == _B - 1)
    def _fin():
        probs = counts_ref[...] * (1.0 / _N)
        ent = -jnp.sum(probs * jnp.log(probs + 1e-10))
        pplx_ref[...] = jnp.exp(ent).reshape(1, 1)


def _make_call(interpret=False):
    return pl.pallas_call(
        _vq_body,
        grid=(_B,),
        in_specs=[
            pl.BlockSpec((1, _D, _T), lambda b: (b, 0, 0)),
            pl.BlockSpec((_D, _C), lambda b: (0, 0)),
        ],
        out_specs=[
            pl.BlockSpec((1, _D, _T), lambda b: (b, 0, 0)),
            pl.BlockSpec((1, 1, _T), lambda b: (b, 0, 0)),
            pl.BlockSpec((1, 1), lambda b: (0, 0)),
        ],
        out_shape=[
            jax.ShapeDtypeStruct((_B, _D, _T), jnp.float32),
            jax.ShapeDtypeStruct((_B, 1, _T), jnp.int32),
            jax.ShapeDtypeStruct((1, 1), jnp.float32),
        ],
        scratch_shapes=[pltpu.VMEM((1, _C), jnp.float32)],
        interpret=interpret,
    )


def kernel(x, embed):
    q, idx3, pplx = _make_call()(x, embed)
    return q, idx3.reshape(_B, _T), pplx[0, 0]


# speedup vs baseline: 2.4309x; 2.0385x over previous
import jax
import jax.numpy as jnp
from jax.experimental import pallas as pl
from jax.experimental.pallas import tpu as pltpu

_D = 64
_C = 1024
_B = 16
_T = 1024
_N = _B * _T


def _vq_body(x_ref, e_ref, q_ref, idx_ref, pplx_ref, counts_ref):
    b = pl.program_id(0)
    xb = x_ref[0]
    e = e_ref[...]
    xe = jax.lax.dot_general(xb.astype(jnp.bfloat16), e.astype(jnp.bfloat16),
                             (((0,), (0,)), ((), ())),
                             preferred_element_type=jnp.float32)
    x2 = jnp.sum(xb * xb, axis=0)
    e2 = jnp.sum(e * e, axis=0)
    dist = (x2[:, None] - 2.0 * xe) + e2[None, :]
    idx = jnp.argmin(dist, axis=1).astype(jnp.int32)
    idx_ref[0, 0] = idx
    oh_bf = (jax.lax.broadcasted_iota(jnp.int32, (_T, _C), 1)
             == idx[:, None]).astype(jnp.bfloat16)
    q = jax.lax.dot_general(e.astype(jnp.bfloat16), oh_bf,
                            (((1,), (1,)), ((), ())),
                            preferred_element_type=jnp.float32)
    q_ref[0] = xb + (q - xb)

    @pl.when(b == 0)
    def _init():
        counts_ref[...] = jnp.zeros_like(counts_ref)

    ones_row = jnp.ones((1, _T), jnp.bfloat16)
    counts_ref[...] += jax.lax.dot_general(ones_row, oh_bf,
                                           (((1,), (0,)), ((), ())),
                                           preferred_element_type=jnp.float32)

    @pl.when(b == _B - 1)
    def _fin():
        probs = counts_ref[...] * (1.0 / _N)
        ent = -jnp.sum(probs * jnp.log(probs + 1e-10))
        pplx_ref[...] = jnp.exp(ent).reshape(1, 1)


def _make_call(interpret=False):
    return pl.pallas_call(
        _vq_body,
        grid=(_B,),
        in_specs=[
            pl.BlockSpec((1, _D, _T), lambda b: (b, 0, 0)),
            pl.BlockSpec((_D, _C), lambda b: (0, 0)),
        ],
        out_specs=[
            pl.BlockSpec((1, _D, _T), lambda b: (b, 0, 0)),
            pl.BlockSpec((1, 1, _T), lambda b: (b, 0, 0)),
            pl.BlockSpec((1, 1), lambda b: (0, 0)),
        ],
        out_shape=[
            jax.ShapeDtypeStruct((_B, _D, _T), jnp.float32),
            jax.ShapeDtypeStruct((_B, 1, _T), jnp.int32),
            jax.ShapeDtypeStruct((1, 1), jnp.float32),
        ],
        scratch_shapes=[pltpu.VMEM((1, _C), jnp.float32)],
        interpret=interpret,
    )


def kernel(x, embed):
    q, idx3, pplx = _make_call()(x, embed)
    return q, idx3.reshape(_B, _T), pplx[0, 0]


# speedup vs baseline: 2.5102x; 1.0326x over previous
import jax
import jax.numpy as jnp
from jax.experimental import pallas as pl
from jax.experimental.pallas import tpu as pltpu

_D = 64
_C = 1024
_B = 16
_T = 1024
_N = _B * _T


def _vq_body(x_ref, e_ref, q_ref, idx_ref, pplx_ref, counts_ref):
    b = pl.program_id(0)
    xb = x_ref[0]
    e = e_ref[...]
    xeT = jax.lax.dot_general(e.astype(jnp.bfloat16), xb.astype(jnp.bfloat16),
                              (((0,), (0,)), ((), ())),
                              preferred_element_type=jnp.float32)
    x2 = jnp.sum(xb * xb, axis=0)
    e2 = jnp.sum(e * e, axis=0)
    e2_col = e2[None, :].T
    dist = (x2[None, :] - 2.0 * xeT) + e2_col
    idx = jnp.argmin(dist, axis=0).astype(jnp.int32)
    idx_ref[0, 0] = idx
    oh_bf = (jax.lax.broadcasted_iota(jnp.int32, (_C, _T), 0)
             == idx[None, :]).astype(jnp.bfloat16)
    q = jax.lax.dot_general(e.astype(jnp.bfloat16), oh_bf,
                            (((1,), (0,)), ((), ())),
                            preferred_element_type=jnp.float32)
    q_ref[0] = xb + (q - xb)

    @pl.when(b == 0)
    def _init():
        counts_ref[...] = jnp.zeros_like(counts_ref)

    ones_row = jnp.ones((1, _T), jnp.bfloat16)
    counts_ref[...] += jax.lax.dot_general(ones_row, oh_bf,
                                           (((1,), (1,)), ((), ())),
                                           preferred_element_type=jnp.float32)

    @pl.when(b == _B - 1)
    def _fin():
        probs = counts_ref[...] * (1.0 / _N)
        ent = -jnp.sum(probs * jnp.log(probs + 1e-10))
        pplx_ref[...] = jnp.exp(ent).reshape(1, 1)


def _make_call(interpret=False):
    return pl.pallas_call(
        _vq_body,
        grid=(_B,),
        in_specs=[
            pl.BlockSpec((1, _D, _T), lambda b: (b, 0, 0)),
            pl.BlockSpec((_D, _C), lambda b: (0, 0)),
        ],
        out_specs=[
            pl.BlockSpec((1, _D, _T), lambda b: (b, 0, 0)),
            pl.BlockSpec((1, 1, _T), lambda b: (b, 0, 0)),
            pl.BlockSpec((1, 1), lambda b: (0, 0)),
        ],
        out_shape=[
            jax.ShapeDtypeStruct((_B, _D, _T), jnp.float32),
            jax.ShapeDtypeStruct((_B, 1, _T), jnp.int32),
            jax.ShapeDtypeStruct((1, 1), jnp.float32),
        ],
        scratch_shapes=[pltpu.VMEM((1, _C), jnp.float32)],
        interpret=interpret,
    )


def kernel(x, embed):
    q, idx3, pplx = _make_call()(x, embed)
    return q, idx3.reshape(_B, _T), pplx[0, 0]


# speedup vs baseline: 2.8304x; 1.1276x over previous
import jax
import jax.numpy as jnp
from jax.experimental import pallas as pl
from jax.experimental.pallas import tpu as pltpu

_D = 64
_C = 1024
_B = 16
_T = 1024
_N = _B * _T
_U = 2


def _vq_body(x_ref, e_ref, q_ref, idx_ref, pplx_ref, counts_ref):
    b = pl.program_id(0)
    e = e_ref[...]
    e_bf = e.astype(jnp.bfloat16)
    e2 = jnp.sum(e * e, axis=0)
    e2_col = e2[None, :].T

    @pl.when(b == 0)
    def _init():
        counts_ref[...] = jnp.zeros_like(counts_ref)

    cnt = jnp.zeros((1, _C), jnp.float32)
    for i in range(_U):
        xb = x_ref[i]
        xeT = jax.lax.dot_general(e_bf, xb.astype(jnp.bfloat16),
                                  (((0,), (0,)), ((), ())),
                                  preferred_element_type=jnp.float32)
        x2 = jnp.sum(xb * xb, axis=0)
        dist = (x2[None, :] - 2.0 * xeT) + e2_col
        idx = jnp.argmin(dist, axis=0).astype(jnp.int32)
        idx_ref[i, 0] = idx
        oh_bf = (jax.lax.broadcasted_iota(jnp.int32, (_C, _T), 0)
                 == idx[None, :]).astype(jnp.bfloat16)
        q = jax.lax.dot_general(e_bf, oh_bf,
                                (((1,), (0,)), ((), ())),
                                preferred_element_type=jnp.float32)
        q_ref[i] = xb + (q - xb)
        ones_row = jnp.ones((1, _T), jnp.bfloat16)
        cnt = cnt + jax.lax.dot_general(ones_row, oh_bf,
                                        (((1,), (1,)), ((), ())),
                                        preferred_element_type=jnp.float32)

    counts_ref[...] += cnt

    @pl.when(b == (_B // _U) - 1)
    def _fin():
        probs = counts_ref[...] * (1.0 / _N)
        ent = -jnp.sum(probs * jnp.log(probs + 1e-10))
        pplx_ref[...] = jnp.exp(ent).reshape(1, 1)


def _make_call(interpret=False):
    return pl.pallas_call(
        _vq_body,
        grid=(_B // _U,),
        in_specs=[
            pl.BlockSpec((_U, _D, _T), lambda b: (b, 0, 0)),
            pl.BlockSpec((_D, _C), lambda b: (0, 0)),
        ],
        out_specs=[
            pl.BlockSpec((_U, _D, _T), lambda b: (b, 0, 0)),
            pl.BlockSpec((_U, 1, _T), lambda b: (b, 0, 0)),
            pl.BlockSpec((1, 1), lambda b: (0, 0)),
        ],
        out_shape=[
            jax.ShapeDtypeStruct((_B, _D, _T), jnp.float32),
            jax.ShapeDtypeStruct((_B, 1, _T), jnp.int32),
            jax.ShapeDtypeStruct((1, 1), jnp.float32),
        ],
        scratch_shapes=[pltpu.VMEM((1, _C), jnp.float32)],
        interpret=interpret,
    )


def kernel(x, embed):
    q, idx3, pplx = _make_call()(x, embed)
    return q, idx3.reshape(_B, _T), pplx[0, 0]


# speedup vs baseline: 2.9158x; 1.0302x over previous
import jax
import jax.numpy as jnp
from jax.experimental import pallas as pl
from jax.experimental.pallas import tpu as pltpu

_D = 64
_C = 1024
_B = 16
_T = 1024
_N = _B * _T
_U = 4


def _vq_body(x_ref, e_ref, q_ref, idx_ref, pplx_ref, counts_ref):
    b = pl.program_id(0)
    e = e_ref[...]
    e_bf = e.astype(jnp.bfloat16)
    e2 = jnp.sum(e * e, axis=0)
    e2_col = e2[None, :].T

    @pl.when(b == 0)
    def _init():
        counts_ref[...] = jnp.zeros_like(counts_ref)

    cnt = jnp.zeros((1, _C), jnp.float32)
    for i in range(_U):
        xb = x_ref[i]
        xeT = jax.lax.dot_general(e_bf, xb.astype(jnp.bfloat16),
                                  (((0,), (0,)), ((), ())),
                                  preferred_element_type=jnp.float32)
        x2 = jnp.sum(xb * xb, axis=0)
        dist = (x2[None, :] - 2.0 * xeT) + e2_col
        idx = jnp.argmin(dist, axis=0).astype(jnp.int32)
        idx_ref[i, 0] = idx
        oh_bf = (jax.lax.broadcasted_iota(jnp.int32, (_C, _T), 0)
                 == idx[None, :]).astype(jnp.bfloat16)
        q = jax.lax.dot_general(e_bf, oh_bf,
                                (((1,), (0,)), ((), ())),
                                preferred_element_type=jnp.float32)
        q_ref[i] = xb + (q - xb)
        ones_row = jnp.ones((1, _T), jnp.bfloat16)
        cnt = cnt + jax.lax.dot_general(ones_row, oh_bf,
                                        (((1,), (1,)), ((), ())),
                                        preferred_element_type=jnp.float32)

    counts_ref[...] += cnt

    @pl.when(b == (_B // _U) - 1)
    def _fin():
        probs = counts_ref[...] * (1.0 / _N)
        ent = -jnp.sum(probs * jnp.log(probs + 1e-10))
        pplx_ref[...] = jnp.exp(ent).reshape(1, 1)


def _make_call(interpret=False):
    return pl.pallas_call(
        _vq_body,
        grid=(_B // _U,),
        in_specs=[
            pl.BlockSpec((_U, _D, _T), lambda b: (b, 0, 0)),
            pl.BlockSpec((_D, _C), lambda b: (0, 0)),
        ],
        out_specs=[
            pl.BlockSpec((_U, _D, _T), lambda b: (b, 0, 0)),
            pl.BlockSpec((_U, 1, _T), lambda b: (b, 0, 0)),
            pl.BlockSpec((1, 1), lambda b: (0, 0)),
        ],
        out_shape=[
            jax.ShapeDtypeStruct((_B, _D, _T), jnp.float32),
            jax.ShapeDtypeStruct((_B, 1, _T), jnp.int32),
            jax.ShapeDtypeStruct((1, 1), jnp.float32),
        ],
        scratch_shapes=[pltpu.VMEM((1, _C), jnp.float32)],
        interpret=interpret,
    )


def kernel(x, embed):
    q, idx3, pplx = _make_call()(x, embed)
    return q, idx3.reshape(_B, _T), pplx[0, 0]


# speedup vs baseline: 2.9188x; 1.0010x over previous
import jax
import jax.numpy as jnp
from jax.experimental import pallas as pl
from jax.experimental.pallas import tpu as pltpu

_D = 64
_C = 1024
_B = 16
_T = 1024
_N = _B * _T
_U = 8


def _vq_body(x_ref, e_ref, q_ref, idx_ref, pplx_ref, counts_ref):
    b = pl.program_id(0)
    e = e_ref[...]
    e_bf = e.astype(jnp.bfloat16)
    e2 = jnp.sum(e * e, axis=0)
    e2_col = e2[None, :].T

    @pl.when(b == 0)
    def _init():
        counts_ref[...] = jnp.zeros_like(counts_ref)

    cnt = jnp.zeros((1, _C), jnp.float32)
    for i in range(_U):
        xb = x_ref[i]
        xeT = jax.lax.dot_general(e_bf, xb.astype(jnp.bfloat16),
                                  (((0,), (0,)), ((), ())),
                                  preferred_element_type=jnp.float32)
        x2 = jnp.sum(xb * xb, axis=0)
        dist = (x2[None, :] - 2.0 * xeT) + e2_col
        idx = jnp.argmin(dist, axis=0).astype(jnp.int32)
        idx_ref[i, 0] = idx
        oh_bf = (jax.lax.broadcasted_iota(jnp.int32, (_C, _T), 0)
                 == idx[None, :]).astype(jnp.bfloat16)
        q = jax.lax.dot_general(e_bf, oh_bf,
                                (((1,), (0,)), ((), ())),
                                preferred_element_type=jnp.float32)
        q_ref[i] = xb + (q - xb)
        ones_row = jnp.ones((1, _T), jnp.bfloat16)
        cnt = cnt + jax.lax.dot_general(ones_row, oh_bf,
                                        (((1,), (1,)), ((), ())),
                                        preferred_element_type=jnp.float32)

    counts_ref[...] += cnt

    @pl.when(b == (_B // _U) - 1)
    def _fin():
        probs = counts_ref[...] * (1.0 / _N)
        ent = -jnp.sum(probs * jnp.log(probs + 1e-10))
        pplx_ref[...] = jnp.exp(ent).reshape(1, 1)


def _make_call(interpret=False):
    return pl.pallas_call(
        _vq_body,
        grid=(_B // _U,),
        in_specs=[
            pl.BlockSpec((_U, _D, _T), lambda b: (b, 0, 0)),
            pl.BlockSpec((_D, _C), lambda b: (0, 0)),
        ],
        out_specs=[
            pl.BlockSpec((_U, _D, _T), lambda b: (b, 0, 0)),
            pl.BlockSpec((_U, 1, _T), lambda b: (b, 0, 0)),
            pl.BlockSpec((1, 1), lambda b: (0, 0)),
        ],
        out_shape=[
            jax.ShapeDtypeStruct((_B, _D, _T), jnp.float32),
            jax.ShapeDtypeStruct((_B, 1, _T), jnp.int32),
            jax.ShapeDtypeStruct((1, 1), jnp.float32),
        ],
        scratch_shapes=[pltpu.VMEM((1, _C), jnp.float32)],
        interpret=interpret,
    )


def kernel(x, embed):
    q, idx3, pplx = _make_call()(x, embed)
    return q, idx3.reshape(_B, _T), pplx[0, 0]
